# Initial kernel scaffold; baseline (speedup 1.0000x reference)
import jax
import jax.numpy as jnp
from jax.experimental import pallas as pl


def kernel(x, edge_index, W1, b1, W2, b2, Ws, bs):
    raise NotImplementedError("write your pallas kernel here")


# speedup vs baseline: 11.3194x; 11.3194x over previous
import functools

import jax
import jax.numpy as jnp
from jax import lax
from jax.experimental import pallas as pl
from jax.experimental.pallas import tpu as pltpu
from jax.experimental.pallas import tpu_sc as plsc

N_NODES_C = 10000
D_C = 128
N_EDGES_C = 320000

_NC = 2
_NS = 16
_NW = _NC * _NS
_EPW = N_EDGES_C // _NW
_EBLK = 80
_NBLK = _EPW // _EBLK
_RPT = N_NODES_C // _NS

_vector_mesh = plsc.VectorSubcoreMesh(
    core_axis_name="core", subcore_axis_name="subcore")


def _deg_body(dst_hbm, out_hbm, idx_v, ones_v, zbuf_v, deg_sh):
    c = lax.axis_index("core")
    s = lax.axis_index("subcore")
    wid = c * _NS + s
    for k in range(_EBLK // 16):
        ones_v[pl.ds(16 * k, 16)] = jnp.ones((16,), jnp.float32)

    @pl.loop(0, 40)
    def _(i):
        zbuf_v[pl.ds(i * 16, 16)] = jnp.zeros((16,), jnp.float32)

    @pl.when(s < _NS - 1)
    def _():
        pltpu.sync_copy(zbuf_v.at[pl.ds(0, 624)],
                        deg_sh.at[pl.ds(s * 624, 624)])
    @pl.when(s == _NS - 1)
    def _():
        pltpu.sync_copy(zbuf_v, deg_sh.at[pl.ds((_NS - 1) * 624, 640)])
    plsc.subcore_barrier()

    base_e = wid * _EPW

    @pl.loop(0, _NBLK)
    def _(j):
        pltpu.sync_copy(dst_hbm.at[pl.ds(base_e + j * _EBLK, _EBLK)], idx_v)
        pltpu.sync_copy(ones_v, deg_sh.at[idx_v], add=True)

    plsc.subcore_barrier()
    @pl.when(s < _NS - 1)
    def _():
        pltpu.sync_copy(deg_sh.at[pl.ds(s * 624, 624)],
                        zbuf_v.at[pl.ds(0, 624)])
        pltpu.sync_copy(zbuf_v.at[pl.ds(0, 624)],
                        out_hbm.at[pl.ds(c * N_NODES_C + s * 624, 624)])
    @pl.when(s == _NS - 1)
    def _():
        pltpu.sync_copy(deg_sh.at[pl.ds((_NS - 1) * 624, 640)], zbuf_v)
        pltpu.sync_copy(zbuf_v,
                        out_hbm.at[pl.ds(c * N_NODES_C + (_NS - 1) * 624, 640)])


def _segsum_body(h_hbm, src_hbm, dst_hbm, out_hbm,
                 src_v, dst_v, rows_v, acc_sh):
    c = lax.axis_index("core")
    s = lax.axis_index("subcore")
    wid = c * _NS + s
    @pl.loop(0, _EBLK)
    def _(i):
        for k in range(D_C // 16):
            rows_v[i, pl.ds(16 * k, 16)] = jnp.zeros((16,), jnp.float32)

    rbase = s * 624

    @pl.when(s < _NS - 1)
    def _():
        for k in range(7):
            pltpu.sync_copy(rows_v, acc_sh.at[pl.ds(rbase + k * _EBLK, _EBLK)])
        pltpu.sync_copy(rows_v.at[pl.ds(0, 64)],
                        acc_sh.at[pl.ds(rbase + 560, 64)])
    @pl.when(s == _NS - 1)
    def _():
        for k in range(8):
            pltpu.sync_copy(rows_v, acc_sh.at[pl.ds(rbase + k * _EBLK, _EBLK)])
    plsc.subcore_barrier()

    base_e = wid * _EPW

    @pl.loop(0, _NBLK)
    def _(j):
        base = base_e + j * _EBLK
        pltpu.sync_copy(src_hbm.at[pl.ds(base, _EBLK)], src_v)
        pltpu.sync_copy(dst_hbm.at[pl.ds(base, _EBLK)], dst_v)
        pltpu.sync_copy(h_hbm.at[src_v], rows_v)
        pltpu.sync_copy(rows_v, acc_sh.at[dst_v], add=True)

    plsc.subcore_barrier()
    @pl.when(s < _NS - 1)
    def _():
        for k in range(7):
            pltpu.sync_copy(acc_sh.at[pl.ds(rbase + k * _EBLK, _EBLK)], rows_v)
            pltpu.sync_copy(rows_v,
                            out_hbm.at[c, pl.ds(rbase + k * _EBLK, _EBLK)])
        pltpu.sync_copy(acc_sh.at[pl.ds(rbase + 560, 64)],
                        rows_v.at[pl.ds(0, 64)])
        pltpu.sync_copy(rows_v.at[pl.ds(0, 64)],
                        out_hbm.at[c, pl.ds(rbase + 560, 64)])
    @pl.when(s == _NS - 1)
    def _():
        for k in range(8):
            pltpu.sync_copy(acc_sh.at[pl.ds(rbase + k * _EBLK, _EBLK)], rows_v)
            pltpu.sync_copy(rows_v,
                            out_hbm.at[c, pl.ds(rbase + k * _EBLK, _EBLK)])


@jax.jit
def _sc_degree(dst):
    k = pl.kernel(
        _deg_body,
        out_type=jax.ShapeDtypeStruct((_NC * N_NODES_C,), jnp.float32),
        mesh=_vector_mesh,
        scratch_types=[
            pltpu.VMEM((_EBLK,), jnp.int32),
            pltpu.VMEM((_EBLK,), jnp.float32),
            pltpu.VMEM((640,), jnp.float32),
            pltpu.VMEM_SHARED((N_NODES_C,), jnp.float32),
        ],
    )
    return k(dst)


@jax.jit
def _sc_segsum(h, src, dst):
    k = pl.kernel(
        _segsum_body,
        out_type=jax.ShapeDtypeStruct((_NC, N_NODES_C, D_C), jnp.float32),
        mesh=_vector_mesh,
        scratch_types=[
            pltpu.VMEM((_EBLK,), jnp.int32),
            pltpu.VMEM((_EBLK,), jnp.int32),
            pltpu.VMEM((_EBLK, D_C), jnp.float32),
            pltpu.VMEM_SHARED((N_NODES_C, D_C), jnp.float32),
        ],
    )
    return k(h, src, dst)


def _dinv(degp_ref):
    deg = degp_ref[:, 0:1] + degp_ref[:, 1:2]
    return jnp.where(deg > 0.0,
                     lax.rsqrt(jnp.maximum(deg, 1e-12)), 0.0)


def _tc1_body(x_ref, w1_ref, degp_ref, hs_ref):
    dinv = _dinv(degp_ref)
    h = jnp.dot(x_ref[...], w1_ref[...],
                preferred_element_type=jnp.float32,
                precision=lax.Precision.HIGHEST)
    hs_ref[...] = h * dinv


def _tc2_body(s1_ref, degp_ref, b1_ref, w2_ref, h1_ref, gs_ref):
    dinv = _dinv(degp_ref)
    a1 = (s1_ref[0] + s1_ref[1]) * dinv + b1_ref[...]
    h1 = jnp.maximum(a1, 0.0)
    h1_ref[...] = h1
    g = jnp.dot(h1, w2_ref[...],
                preferred_element_type=jnp.float32,
                precision=lax.Precision.HIGHEST)
    gs_ref[...] = g * dinv


def _tc3_body(s2_ref, degp_ref, b2_ref, h1_ref, ws_ref, bs_ref, out_ref):
    dinv = _dinv(degp_ref)
    a2 = (s2_ref[0] + s2_ref[1]) * dinv + b2_ref[...]
    h2 = jnp.maximum(a2, 0.0) + h1_ref[...]
    out_ref[...] = jnp.dot(h2, ws_ref[...],
                           preferred_element_type=jnp.float32,
                           precision=lax.Precision.HIGHEST) + bs_ref[...]


_f32 = jnp.float32


def _tc1(x, W1, degp):
    return pl.pallas_call(
        _tc1_body,
        out_shape=jax.ShapeDtypeStruct((N_NODES_C, D_C), _f32),
    )(x, W1, degp)


def _tc2(s1, degp, b1, W2):
    return pl.pallas_call(
        _tc2_body,
        out_shape=(jax.ShapeDtypeStruct((N_NODES_C, D_C), _f32),
                   jax.ShapeDtypeStruct((N_NODES_C, D_C), _f32)),
    )(s1, degp, b1, W2)


def _tc3(s2, degp, b2, h1, Ws, bs):
    return pl.pallas_call(
        _tc3_body,
        out_shape=jax.ShapeDtypeStruct((N_NODES_C, D_C), _f32),
    )(s2, degp, b2, h1, Ws, bs)


@jax.jit
def kernel(x, edge_index, W1, b1, W2, b2, Ws, bs):
    src = edge_index[0]
    dst = edge_index[1]

    degp = _sc_degree(dst).reshape(_NC, N_NODES_C).T
    hs = _tc1(x, W1, degp)
    s1 = _sc_segsum(hs, src, dst)
    h1, gs = _tc2(s1, degp, b1, W2)
    s2 = _sc_segsum(gs, src, dst)
    return _tc3(s2, degp, b2, h1, Ws, bs)


# speedup vs baseline: 21.8442x; 1.9298x over previous
import functools

import jax
import jax.numpy as jnp
from jax import lax
from jax.experimental import pallas as pl
from jax.experimental.pallas import tpu as pltpu
from jax.experimental.pallas import tpu_sc as plsc

N_NODES_C = 10000
D_C = 128
N_EDGES_C = 320000

_NC = 2
_NS = 16
_NW = _NC * _NS
_EPW = N_EDGES_C // _NW
_EBLK = 80
_NBLK = _EPW // _EBLK
_RPT = N_NODES_C // _NS

_vector_mesh = plsc.VectorSubcoreMesh(
    core_axis_name="core", subcore_axis_name="subcore")


def _deg_body(dst_hbm, out_hbm, idx_v, ones_v, zbuf_v, deg_sh):
    c = lax.axis_index("core")
    s = lax.axis_index("subcore")
    wid = c * _NS + s
    for k in range(_EBLK // 16):
        ones_v[pl.ds(16 * k, 16)] = jnp.ones((16,), jnp.float32)

    @pl.loop(0, 40)
    def _(i):
        zbuf_v[pl.ds(i * 16, 16)] = jnp.zeros((16,), jnp.float32)

    @pl.when(s < _NS - 1)
    def _():
        pltpu.sync_copy(zbuf_v.at[pl.ds(0, 624)],
                        deg_sh.at[pl.ds(s * 624, 624)])
    @pl.when(s == _NS - 1)
    def _():
        pltpu.sync_copy(zbuf_v, deg_sh.at[pl.ds((_NS - 1) * 624, 640)])
    plsc.subcore_barrier()

    base_e = wid * _EPW

    @pl.loop(0, _NBLK)
    def _(j):
        pltpu.sync_copy(dst_hbm.at[pl.ds(base_e + j * _EBLK, _EBLK)], idx_v)
        pltpu.sync_copy(ones_v, deg_sh.at[idx_v], add=True)

    plsc.subcore_barrier()
    @pl.when(s < _NS - 1)
    def _():
        pltpu.sync_copy(deg_sh.at[pl.ds(s * 624, 624)],
                        zbuf_v.at[pl.ds(0, 624)])
        pltpu.sync_copy(zbuf_v.at[pl.ds(0, 624)],
                        out_hbm.at[pl.ds(c * N_NODES_C + s * 624, 624)])
    @pl.when(s == _NS - 1)
    def _():
        pltpu.sync_copy(deg_sh.at[pl.ds((_NS - 1) * 624, 640)], zbuf_v)
        pltpu.sync_copy(zbuf_v,
                        out_hbm.at[pl.ds(c * N_NODES_C + (_NS - 1) * 624, 640)])


_BLK = 128
_NBLK_TOT = N_EDGES_C // _BLK
_NBLK_LO = _NBLK_TOT // _NW
_NBLK_XTRA = _NBLK_TOT - _NBLK_LO * _NW


def _segsum_body(h_hbm, ei_hbm, out_hbm,
                 ibuf_v, rows_v, isems, ssems, acc_sh):
    c = lax.axis_index("core")
    s = lax.axis_index("subcore")
    w = c * _NS + s
    nblk = _NBLK_LO + jnp.where(w < _NBLK_XTRA, 1, 0)

    def idx_start(t):
        pltpu.make_async_copy(
            ei_hbm.at[pl.ds(0, 2), pl.ds((w + _NW * t) * _BLK, _BLK)],
            ibuf_v.at[t & 3], isems.at[t & 3]).start()

    def idx_wait(t):
        pltpu.make_async_copy(
            ei_hbm.at[pl.ds(0, 2), pl.ds(0, _BLK)],
            ibuf_v.at[t & 3], isems.at[t & 3]).wait()

    def gather(t):
        pltpu.sync_copy(h_hbm.at[ibuf_v.at[t & 3, 0]], rows_v.at[t & 1])

    def scat_start(t):
        pltpu.make_async_copy(
            rows_v.at[t & 1], acc_sh.at[ibuf_v.at[t & 3, 1]],
            ssems.at[t & 1]).start(add=True)

    def scat_wait(t):
        pltpu.make_async_copy(
            rows_v.at[t & 1], acc_sh.at[ibuf_v.at[t & 3, 1]],
            ssems.at[t & 1]).wait()

    idx_start(0)
    idx_start(1)

    z_v = rows_v.at[0]

    @pl.loop(0, _BLK)
    def _(i):
        for k in range(D_C // 16):
            z_v[i, pl.ds(16 * k, 16)] = jnp.zeros((16,), jnp.float32)

    rbase = s * 624

    @pl.when(s < _NS - 1)
    def _():
        for k in range(4):
            pltpu.sync_copy(z_v, acc_sh.at[pl.ds(rbase + k * _BLK, _BLK)])
        pltpu.sync_copy(z_v.at[pl.ds(0, 112)],
                        acc_sh.at[pl.ds(rbase + 512, 112)])
    @pl.when(s == _NS - 1)
    def _():
        for k in range(5):
            pltpu.sync_copy(z_v, acc_sh.at[pl.ds(rbase + k * _BLK, _BLK)])
    plsc.subcore_barrier()

    @pl.loop(0, _NBLK_LO)
    def _(t):
        @pl.when(t >= 2)
        def _():
            scat_wait(t - 2)
        @pl.when(t + 2 < nblk)
        def _():
            idx_start(t + 2)
        idx_wait(t)
        gather(t)
        scat_start(t)

    @pl.when(w < _NBLK_XTRA)
    def _():
        t = _NBLK_LO
        scat_wait(t - 2)
        idx_wait(t)
        gather(t)
        scat_start(t)
    scat_wait(nblk - 2)
    scat_wait(nblk - 1)

    plsc.subcore_barrier()
    @pl.when(s < _NS - 1)
    def _():
        for k in range(4):
            pltpu.sync_copy(acc_sh.at[pl.ds(rbase + k * _BLK, _BLK)], z_v)
            pltpu.sync_copy(z_v,
                            out_hbm.at[c, pl.ds(rbase + k * _BLK, _BLK)])
        pltpu.sync_copy(acc_sh.at[pl.ds(rbase + 512, 112)],
                        z_v.at[pl.ds(0, 112)])
        pltpu.sync_copy(z_v.at[pl.ds(0, 112)],
                        out_hbm.at[c, pl.ds(rbase + 512, 112)])
    @pl.when(s == _NS - 1)
    def _():
        for k in range(5):
            pltpu.sync_copy(acc_sh.at[pl.ds(rbase + k * _BLK, _BLK)], z_v)
            pltpu.sync_copy(z_v,
                            out_hbm.at[c, pl.ds(rbase + k * _BLK, _BLK)])


@jax.jit
def _sc_degree(dst):
    k = pl.kernel(
        _deg_body,
        out_type=jax.ShapeDtypeStruct((_NC * N_NODES_C,), jnp.float32),
        mesh=_vector_mesh,
        scratch_types=[
            pltpu.VMEM((_EBLK,), jnp.int32),
            pltpu.VMEM((_EBLK,), jnp.float32),
            pltpu.VMEM((640,), jnp.float32),
            pltpu.VMEM_SHARED((N_NODES_C,), jnp.float32),
        ],
    )
    return k(dst)


@jax.jit
def _sc_segsum(h, ei):
    k = pl.kernel(
        _segsum_body,
        out_type=jax.ShapeDtypeStruct((_NC, N_NODES_C, D_C), jnp.float32),
        mesh=_vector_mesh,
        scratch_types=[
            pltpu.VMEM((4, 2, _BLK), jnp.int32),
            pltpu.VMEM((2, _BLK, D_C), jnp.float32),
            pltpu.SemaphoreType.DMA((4,)),
            pltpu.SemaphoreType.DMA((2,)),
            pltpu.VMEM_SHARED((N_NODES_C, D_C), jnp.float32),
        ],
    )
    return k(h, ei)


def _dinv(degp_ref):
    deg = degp_ref[:, 0:1] + degp_ref[:, 1:2]
    return jnp.where(deg > 0.0,
                     lax.rsqrt(jnp.maximum(deg, 1e-12)), 0.0)


def _tc1_body(x_ref, w1_ref, degp_ref, hs_ref):
    dinv = _dinv(degp_ref)
    h = jnp.dot(x_ref[...], w1_ref[...],
                preferred_element_type=jnp.float32,
                precision=lax.Precision.HIGHEST)
    hs_ref[...] = h * dinv


def _tc2_body(s1_ref, degp_ref, b1_ref, w2_ref, h1_ref, gs_ref):
    dinv = _dinv(degp_ref)
    a1 = (s1_ref[0] + s1_ref[1]) * dinv + b1_ref[...]
    h1 = jnp.maximum(a1, 0.0)
    h1_ref[...] = h1
    g = jnp.dot(h1, w2_ref[...],
                preferred_element_type=jnp.float32,
                precision=lax.Precision.HIGHEST)
    gs_ref[...] = g * dinv


def _tc3_body(s2_ref, degp_ref, b2_ref, h1_ref, ws_ref, bs_ref, out_ref):
    dinv = _dinv(degp_ref)
    a2 = (s2_ref[0] + s2_ref[1]) * dinv + b2_ref[...]
    h2 = jnp.maximum(a2, 0.0) + h1_ref[...]
    out_ref[...] = jnp.dot(h2, ws_ref[...],
                           preferred_element_type=jnp.float32,
                           precision=lax.Precision.HIGHEST) + bs_ref[...]


_f32 = jnp.float32


def _tc1(x, W1, degp):
    return pl.pallas_call(
        _tc1_body,
        out_shape=jax.ShapeDtypeStruct((N_NODES_C, D_C), _f32),
    )(x, W1, degp)


def _tc2(s1, degp, b1, W2):
    return pl.pallas_call(
        _tc2_body,
        out_shape=(jax.ShapeDtypeStruct((N_NODES_C, D_C), _f32),
                   jax.ShapeDtypeStruct((N_NODES_C, D_C), _f32)),
    )(s1, degp, b1, W2)


def _tc3(s2, degp, b2, h1, Ws, bs):
    return pl.pallas_call(
        _tc3_body,
        out_shape=jax.ShapeDtypeStruct((N_NODES_C, D_C), _f32),
    )(s2, degp, b2, h1, Ws, bs)


@jax.jit
def kernel(x, edge_index, W1, b1, W2, b2, Ws, bs):
    dst = edge_index[1]

    degp = _sc_degree(dst).reshape(_NC, N_NODES_C).T
    hs = _tc1(x, W1, degp)
    s1 = _sc_segsum(hs, edge_index)
    h1, gs = _tc2(s1, degp, b1, W2)
    s2 = _sc_segsum(gs, edge_index)
    return _tc3(s2, degp, b2, h1, Ws, bs)


# speedup vs baseline: 26.1400x; 1.1967x over previous
import functools

import jax
import jax.numpy as jnp
from jax import lax
from jax.experimental import pallas as pl
from jax.experimental.pallas import tpu as pltpu
from jax.experimental.pallas import tpu_sc as plsc

N_NODES_C = 10000
D_C = 128
N_EDGES_C = 320000

_NC = 2
_NS = 16
_NW = _NC * _NS
_EPW = N_EDGES_C // _NW
_EBLK = 80
_NBLK = _EPW // _EBLK
_RPT = N_NODES_C // _NS

_vector_mesh = plsc.VectorSubcoreMesh(
    core_axis_name="core", subcore_axis_name="subcore")


def _deg_body(ei_hbm, out_hbm, ibuf_v, ones_v, zbuf_v, isems, ssems, deg_sh):
    c = lax.axis_index("core")
    s = lax.axis_index("subcore")
    w = c * _NS + s
    nblk = _NBLK_LO + jnp.where(w < _NBLK_XTRA, 1, 0)

    def idx_start(t):
        pltpu.make_async_copy(
            ei_hbm.at[pl.ds(0, 2), pl.ds((w + _NW * t) * _BLK, _BLK)],
            ibuf_v.at[t & 3], isems.at[t & 3]).start()

    def idx_wait(t):
        pltpu.make_async_copy(
            ei_hbm.at[pl.ds(0, 2), pl.ds(0, _BLK)],
            ibuf_v.at[t & 3], isems.at[t & 3]).wait()

    def scat_start(t):
        pltpu.make_async_copy(
            ones_v, deg_sh.at[ibuf_v.at[t & 3, 1]],
            ssems.at[t & 1]).start(add=True)

    def scat_wait(t):
        pltpu.make_async_copy(
            ones_v, deg_sh.at[ibuf_v.at[t & 3, 1]],
            ssems.at[t & 1]).wait()

    idx_start(0)
    idx_start(1)

    for k in range(_BLK // 16):
        ones_v[pl.ds(16 * k, 16)] = jnp.ones((16,), jnp.float32)

    @pl.loop(0, 40)
    def _(i):
        zbuf_v[pl.ds(i * 16, 16)] = jnp.zeros((16,), jnp.float32)

    @pl.when(s < _NS - 1)
    def _():
        pltpu.sync_copy(zbuf_v.at[pl.ds(0, 624)],
                        deg_sh.at[pl.ds(s * 624, 624)])
    @pl.when(s == _NS - 1)
    def _():
        pltpu.sync_copy(zbuf_v, deg_sh.at[pl.ds((_NS - 1) * 624, 640)])
    plsc.subcore_barrier()

    @pl.loop(0, _NBLK_LO)
    def _(t):
        @pl.when(t >= 2)
        def _():
            scat_wait(t - 2)
        @pl.when(t + 2 < nblk)
        def _():
            idx_start(t + 2)
        idx_wait(t)
        scat_start(t)

    @pl.when(w < _NBLK_XTRA)
    def _():
        t = _NBLK_LO
        scat_wait(t - 2)
        idx_wait(t)
        scat_start(t)
    scat_wait(nblk - 2)
    scat_wait(nblk - 1)

    plsc.subcore_barrier()
    @pl.when(s < _NS - 1)
    def _():
        pltpu.sync_copy(deg_sh.at[pl.ds(s * 624, 624)],
                        zbuf_v.at[pl.ds(0, 624)])
        pltpu.sync_copy(zbuf_v.at[pl.ds(0, 624)],
                        out_hbm.at[pl.ds(c * N_NODES_C + s * 624, 624)])
    @pl.when(s == _NS - 1)
    def _():
        pltpu.sync_copy(deg_sh.at[pl.ds((_NS - 1) * 624, 640)], zbuf_v)
        pltpu.sync_copy(zbuf_v,
                        out_hbm.at[pl.ds(c * N_NODES_C + (_NS - 1) * 624, 640)])


_BLK = 128
_NBLK_TOT = N_EDGES_C // _BLK
_NBLK_LO = _NBLK_TOT // _NW
_NBLK_XTRA = _NBLK_TOT - _NBLK_LO * _NW


def _segsum_body(h_hbm, ei_hbm, out_hbm,
                 ibuf_v, rows_v, isems, ssems, acc_sh):
    c = lax.axis_index("core")
    s = lax.axis_index("subcore")
    w = c * _NS + s
    nblk = _NBLK_LO + jnp.where(w < _NBLK_XTRA, 1, 0)

    def idx_start(t):
        pltpu.make_async_copy(
            ei_hbm.at[pl.ds(0, 2), pl.ds((w + _NW * t) * _BLK, _BLK)],
            ibuf_v.at[t & 3], isems.at[t & 3]).start()

    def idx_wait(t):
        pltpu.make_async_copy(
            ei_hbm.at[pl.ds(0, 2), pl.ds(0, _BLK)],
            ibuf_v.at[t & 3], isems.at[t & 3]).wait()

    def gather(t):
        pltpu.sync_copy(h_hbm.at[ibuf_v.at[t & 3, 0]], rows_v.at[t & 1])

    def scat_start(t):
        pltpu.make_async_copy(
            rows_v.at[t & 1], acc_sh.at[ibuf_v.at[t & 3, 1]],
            ssems.at[t & 1]).start(add=True)

    def scat_wait(t):
        pltpu.make_async_copy(
            rows_v.at[t & 1], acc_sh.at[ibuf_v.at[t & 3, 1]],
            ssems.at[t & 1]).wait()

    idx_start(0)
    idx_start(1)

    z_v = rows_v.at[0]

    @pl.loop(0, _BLK)
    def _(i):
        for k in range(D_C // 16):
            z_v[i, pl.ds(16 * k, 16)] = jnp.zeros((16,), jnp.float32)

    rbase = s * 624

    @pl.when(s < _NS - 1)
    def _():
        for k in range(4):
            pltpu.sync_copy(z_v, acc_sh.at[pl.ds(rbase + k * _BLK, _BLK)])
        pltpu.sync_copy(z_v.at[pl.ds(0, 112)],
                        acc_sh.at[pl.ds(rbase + 512, 112)])
    @pl.when(s == _NS - 1)
    def _():
        for k in range(5):
            pltpu.sync_copy(z_v, acc_sh.at[pl.ds(rbase + k * _BLK, _BLK)])
    plsc.subcore_barrier()

    @pl.loop(0, _NBLK_LO)
    def _(t):
        @pl.when(t >= 2)
        def _():
            scat_wait(t - 2)
        @pl.when(t + 2 < nblk)
        def _():
            idx_start(t + 2)
        idx_wait(t)
        gather(t)
        scat_start(t)

    @pl.when(w < _NBLK_XTRA)
    def _():
        t = _NBLK_LO
        scat_wait(t - 2)
        idx_wait(t)
        gather(t)
        scat_start(t)
    scat_wait(nblk - 2)
    scat_wait(nblk - 1)

    plsc.subcore_barrier()
    @pl.when(s < _NS - 1)
    def _():
        for k in range(4):
            pltpu.sync_copy(acc_sh.at[pl.ds(rbase + k * _BLK, _BLK)], z_v)
            pltpu.sync_copy(z_v,
                            out_hbm.at[c, pl.ds(rbase + k * _BLK, _BLK)])
        pltpu.sync_copy(acc_sh.at[pl.ds(rbase + 512, 112)],
                        z_v.at[pl.ds(0, 112)])
        pltpu.sync_copy(z_v.at[pl.ds(0, 112)],
                        out_hbm.at[c, pl.ds(rbase + 512, 112)])
    @pl.when(s == _NS - 1)
    def _():
        for k in range(5):
            pltpu.sync_copy(acc_sh.at[pl.ds(rbase + k * _BLK, _BLK)], z_v)
            pltpu.sync_copy(z_v,
                            out_hbm.at[c, pl.ds(rbase + k * _BLK, _BLK)])


@jax.jit
def _sc_degree(ei):
    k = pl.kernel(
        _deg_body,
        out_type=jax.ShapeDtypeStruct((_NC * N_NODES_C,), jnp.float32),
        mesh=_vector_mesh,
        scratch_types=[
            pltpu.VMEM((4, 2, _BLK), jnp.int32),
            pltpu.VMEM((_BLK,), jnp.float32),
            pltpu.VMEM((640,), jnp.float32),
            pltpu.SemaphoreType.DMA((4,)),
            pltpu.SemaphoreType.DMA((2,)),
            pltpu.VMEM_SHARED((N_NODES_C,), jnp.float32),
        ],
    )
    return k(ei)


@jax.jit
def _sc_segsum(h, ei):
    k = pl.kernel(
        _segsum_body,
        out_type=jax.ShapeDtypeStruct((_NC, N_NODES_C, D_C), jnp.float32),
        mesh=_vector_mesh,
        scratch_types=[
            pltpu.VMEM((4, 2, _BLK), jnp.int32),
            pltpu.VMEM((2, _BLK, D_C), jnp.float32),
            pltpu.SemaphoreType.DMA((4,)),
            pltpu.SemaphoreType.DMA((2,)),
            pltpu.VMEM_SHARED((N_NODES_C, D_C), jnp.float32),
        ],
    )
    return k(h, ei)


def _dinv(degp_ref):
    deg = degp_ref[:, 0:1] + degp_ref[:, 1:2]
    return jnp.where(deg > 0.0,
                     lax.rsqrt(jnp.maximum(deg, 1e-12)), 0.0)


def _tc1_body(x_ref, w1_ref, degp_ref, hs_ref):
    dinv = _dinv(degp_ref)
    h = jnp.dot(x_ref[...], w1_ref[...],
                preferred_element_type=jnp.float32,
                precision=lax.Precision.HIGHEST)
    hs_ref[...] = h * dinv


def _tc2_body(s1_ref, degp_ref, b1_ref, w2_ref, h1_ref, gs_ref):
    dinv = _dinv(degp_ref)
    a1 = (s1_ref[0] + s1_ref[1]) * dinv + b1_ref[...]
    h1 = jnp.maximum(a1, 0.0)
    h1_ref[...] = h1
    g = jnp.dot(h1, w2_ref[...],
                preferred_element_type=jnp.float32,
                precision=lax.Precision.HIGHEST)
    gs_ref[...] = g * dinv


def _tc3_body(s2_ref, degp_ref, b2_ref, h1_ref, ws_ref, bs_ref, out_ref):
    dinv = _dinv(degp_ref)
    a2 = (s2_ref[0] + s2_ref[1]) * dinv + b2_ref[...]
    h2 = jnp.maximum(a2, 0.0) + h1_ref[...]
    out_ref[...] = jnp.dot(h2, ws_ref[...],
                           preferred_element_type=jnp.float32,
                           precision=lax.Precision.HIGHEST) + bs_ref[...]


_f32 = jnp.float32


def _tc1(x, W1, degp):
    return pl.pallas_call(
        _tc1_body,
        out_shape=jax.ShapeDtypeStruct((N_NODES_C, D_C), _f32),
    )(x, W1, degp)


def _tc2(s1, degp, b1, W2):
    return pl.pallas_call(
        _tc2_body,
        out_shape=(jax.ShapeDtypeStruct((N_NODES_C, D_C), _f32),
                   jax.ShapeDtypeStruct((N_NODES_C, D_C), _f32)),
    )(s1, degp, b1, W2)


def _tc3(s2, degp, b2, h1, Ws, bs):
    return pl.pallas_call(
        _tc3_body,
        out_shape=jax.ShapeDtypeStruct((N_NODES_C, D_C), _f32),
    )(s2, degp, b2, h1, Ws, bs)


@jax.jit
def kernel(x, edge_index, W1, b1, W2, b2, Ws, bs):
    degp = _sc_degree(edge_index).reshape(_NC, N_NODES_C).T
    hs = _tc1(x, W1, degp)
    s1 = _sc_segsum(hs, edge_index)
    h1, gs = _tc2(s1, degp, b1, W2)
    s2 = _sc_segsum(gs, edge_index)
    return _tc3(s2, degp, b2, h1, Ws, bs)
